# baseline (speedup 1.0000x reference)
import functools

import jax
import jax.numpy as jnp
from jax import lax
from jax.experimental import pallas as pl
from jax.experimental.pallas import tpu as pltpu
from jax.experimental.pallas import tpu_sc as plsc

V = 10000
E = 320000
D = 128
D_EDGE = 16

NC = 2
NS = 16
NW = NC * NS
CHUNK = 80
NISLOT = 5
UNROLL = 10
NCH_A = 64
NCH_B = 61
E_A = NW * NCH_A * CHUNK
E_B = NW * NCH_B * CHUNK
VPAD = 10240
VPS = VPAD // NS

NODE_BLK = 1000
EDGE_BLK = 4000


def _node_mm_body(x_ref, win_ref, wn_ref, ws_ref, hv_ref, hs_ref):
    h = jnp.maximum(
        jnp.dot(x_ref[...], win_ref[...], preferred_element_type=jnp.float32), 0.0
    )
    hv_ref[...] = jnp.dot(h, wn_ref[...], preferred_element_type=jnp.float32)
    hs_ref[...] = jnp.dot(h, ws_ref[...], preferred_element_type=jnp.float32)


def _node_mm(x, w_in, w_n, w_s):
    return pl.pallas_call(
        _node_mm_body,
        grid=(V // NODE_BLK,),
        in_specs=[
            pl.BlockSpec((NODE_BLK, D), lambda i: (i, 0)),
            pl.BlockSpec((D, D), lambda i: (0, 0)),
            pl.BlockSpec((D, D), lambda i: (0, 0)),
            pl.BlockSpec((D, D), lambda i: (0, 0)),
        ],
        out_specs=[
            pl.BlockSpec((NODE_BLK, D), lambda i: (i, 0)),
            pl.BlockSpec((NODE_BLK, D), lambda i: (i, 0)),
        ],
        out_shape=[
            jax.ShapeDtypeStruct((V, D), jnp.float32),
            jax.ShapeDtypeStruct((V, D), jnp.float32),
        ],
    )(x, w_in, w_n, w_s)


def _edge_mm_body(ef2_ref, wlo_ref, whi_ref, he2_ref):
    lo = jnp.dot(ef2_ref[...], wlo_ref[...], preferred_element_type=jnp.float32)
    hi = jnp.dot(ef2_ref[...], whi_ref[...], preferred_element_type=jnp.float32)
    au = jax.lax.bitcast_convert_type(
        lo.astype(jnp.bfloat16), jnp.uint16).astype(jnp.int32)
    bu = jax.lax.bitcast_convert_type(
        hi.astype(jnp.bfloat16), jnp.uint16).astype(jnp.int32)
    he2_ref[...] = au | (bu << 16)


def _edge_mm(ef2, w_lo, w_hi, blk):
    rows = ef2.shape[0]
    return pl.pallas_call(
        _edge_mm_body,
        grid=(rows // blk,),
        in_specs=[
            pl.BlockSpec((blk, 2 * D_EDGE), lambda i: (i, 0)),
            pl.BlockSpec((2 * D_EDGE, D), lambda i: (0, 0)),
            pl.BlockSpec((2 * D_EDGE, D), lambda i: (0, 0)),
        ],
        out_specs=pl.BlockSpec((blk, D), lambda i: (i, 0)),
        out_shape=jax.ShapeDtypeStruct((rows, D), jnp.int32),
    )(ef2, w_lo, w_hi)


def _packed_edge_weights(w_e):
    cols = jnp.arange(D).reshape(D // 32, 2, 16)
    w_a = w_e[:, cols[:, 0, :].reshape(-1)]
    w_b = w_e[:, cols[:, 1, :].reshape(-1)]
    z = jnp.zeros((D_EDGE, D // 2), jnp.float32)
    w_lo = jnp.concatenate(
        [jnp.concatenate([w_a, z], 1), jnp.concatenate([z, w_a], 1)], 0)
    w_hi = jnp.concatenate(
        [jnp.concatenate([w_b, z], 1), jnp.concatenate([z, w_b], 1)], 0)
    return w_lo, w_hi


def _edge_sc_body(nch, hv_hbm, idx_hbm, he2_hbm, init_hbm, out_hbm,
                  idxv, grows, erows, acc, si0, si1, si2, si3, si4,
                  sg0, sg1, sh0, sh1):
    c = lax.axis_index("c")
    s = lax.axis_index("s")
    wid = c * NS + s
    ept = nch * CHUNK

    zbase = s * VPS
    pltpu.sync_copy(init_hbm.at[c, pl.ds(zbase, VPS)],
                    acc.at[pl.ds(zbase, VPS)])
    plsc.subcore_barrier()

    ebase = wid * ept
    ebaseh = wid * (ept // 2)
    sis = (si0, si1, si2, si3, si4)
    sgs = (sg0, sg1)
    shs = (sh0, sh1)

    def _issue_idx(g, i):
        pltpu.async_copy(idx_hbm.at[wid, g], idxv.at[i], sis[i])

    def _wait_idx(i):
        pltpu.make_async_copy(idx_hbm.at[0, 0], idxv.at[i], sis[i]).wait()

    def _issue_data(g, i, b):
        pltpu.async_copy(hv_hbm.at[idxv.at[i, 0]], grows.at[b], sgs[b])
        pltpu.async_copy(
            he2_hbm.at[pl.ds(ebaseh + g * (CHUNK // 2), CHUNK // 2)],
            erows.at[b], shs[b])

    def _drain_data(b):
        pltpu.make_async_copy(hv_hbm.at[idxv.at[0, 0]], grows.at[b],
                              sgs[b]).wait()
        pltpu.make_async_copy(
            he2_hbm.at[pl.ds(ebaseh, CHUNK // 2)], erows.at[b],
            shs[b]).wait()

    def _mul_scatter(i, b):
        @plsc.parallel_loop(0, CHUNK // 2)
        def _mul(rp):
            for h in range(2):
                r = 2 * rp + h
                for m in range(D // 32):
                    w = erows[b, rp, pl.ds(h * 64 + m * 16, 16)]
                    af = jax.lax.bitcast_convert_type(w << 16, jnp.float32)
                    bf = jax.lax.bitcast_convert_type(
                        w & jnp.int32(-65536), jnp.float32)
                    sl0 = pl.ds((2 * m) * 16, 16)
                    sl1 = pl.ds((2 * m + 1) * 16, 16)
                    grows[b, r, sl0] = grows[b, r, sl0] * af
                    grows[b, r, sl1] = grows[b, r, sl1] * bf

        pltpu.sync_copy(grows.at[b], acc.at[idxv.at[i, 1]], add=True)

    tail = nch % UNROLL

    def _phase(g, base):
        i, b = g % NISLOT, g % 2
        if g + 1 < nch:
            _wait_idx((g + 1) % NISLOT)
            _issue_data(base + (g + 1), (g + 1) % NISLOT, (g + 1) % 2)
        _drain_data(b)
        _mul_scatter(i, b)
        if g + 4 < nch:
            _issue_idx(base + (g + 4), (g + 4) % NISLOT)

    for g in range(4):
        _issue_idx(g, g)
    _wait_idx(0)
    _issue_data(0, 0, 0)

    @pl.loop(0, (nch - tail) // UNROLL - 1)
    def _block(blk):
        base = blk * UNROLL
        for k in range(UNROLL):
            _phase(k, base)

    for g in range(nch - tail - UNROLL, nch):
        _phase(g, 0)

    plsc.subcore_barrier()
    pltpu.sync_copy(acc.at[pl.ds(zbase, VPS)], out_hbm.at[c, pl.ds(zbase, VPS)])


def _edge_sc(nch, hv, idx, he2, init):
    mesh = plsc.VectorSubcoreMesh(
        core_axis_name="c", subcore_axis_name="s", num_cores=NC, num_subcores=NS
    )
    return pl.kernel(
        functools.partial(_edge_sc_body, nch),
        out_type=jax.ShapeDtypeStruct((NC, VPAD, D), jnp.float32),
        mesh=mesh,
        scratch_types=[
            pltpu.VMEM((NISLOT, 2, CHUNK), jnp.int32),
            pltpu.VMEM((2, CHUNK, D), jnp.float32),
            pltpu.VMEM((2, CHUNK // 2, D), jnp.int32),
            pltpu.VMEM_SHARED((VPAD, D), jnp.float32),
            pltpu.SemaphoreType.DMA,
            pltpu.SemaphoreType.DMA,
            pltpu.SemaphoreType.DMA,
            pltpu.SemaphoreType.DMA,
            pltpu.SemaphoreType.DMA,
            pltpu.SemaphoreType.DMA,
            pltpu.SemaphoreType.DMA,
            pltpu.SemaphoreType.DMA,
            pltpu.SemaphoreType.DMA,
        ],
    )(hv, idx, he2, init)


def _combine_body(p_ref, hs_ref, out_ref):
    out_ref[...] = (p_ref[0] + p_ref[1]) * hs_ref[...]


def _combine(partials, h_self):
    return pl.pallas_call(
        _combine_body,
        grid=(V // NODE_BLK,),
        in_specs=[
            pl.BlockSpec((NC, NODE_BLK, D), lambda i: (0, i, 0)),
            pl.BlockSpec((NODE_BLK, D), lambda i: (i, 0)),
        ],
        out_specs=pl.BlockSpec((NODE_BLK, D), lambda i: (i, 0)),
        out_shape=jax.ShapeDtypeStruct((V, D), jnp.float32),
    )(partials, h_self)


def kernel(node_feats, edge_index, edge_feats, W_in, W_cm, b_cm, W_e, W_n, W_s):
    src_a = edge_index[0, :E_A]
    dst_a = edge_index[1, :E_A]
    src_b = edge_index[0, E_A:]
    dst_b = edge_index[1, E_A:]
    idx_a = jnp.stack([src_a.reshape(NW, NCH_A, CHUNK),
                       dst_a.reshape(NW, NCH_A, CHUNK)], axis=2)
    idx_b = jnp.stack([src_b.reshape(NW, NCH_B, CHUNK),
                       dst_b.reshape(NW, NCH_B, CHUNK)], axis=2)
    hv, h_self = _node_mm(node_feats, W_in, W_n, W_s)
    w_lo, w_hi = _packed_edge_weights(W_e)
    ef2 = edge_feats.reshape(E // 2, 2 * D_EDGE)
    he2_a = _edge_mm(ef2[:E_A // 2], w_lo, w_hi, 2048)
    he2_b = _edge_mm(ef2[E_A // 2:], w_lo, w_hi, 2440)
    zeros = jnp.zeros((NC, VPAD, D), jnp.float32)
    partials_a = _edge_sc(NCH_A, hv, idx_a, he2_a, zeros)
    partials = _edge_sc(NCH_B, hv, idx_b, he2_b, partials_a)
    return _combine(partials, h_self)


# speedup vs baseline: 1.0959x; 1.0959x over previous
import functools

import jax
import jax.numpy as jnp
from jax import lax
from jax.experimental import pallas as pl
from jax.experimental.pallas import tpu as pltpu
from jax.experimental.pallas import tpu_sc as plsc

V = 10000
E = 320000
D = 128
D_EDGE = 16

NC = 2
NS = 16
NW = NC * NS
CHUNK = 80
NCHUNK = E // (NW * CHUNK)
NISLOT = 5
UNROLL = 10
VPAD = 10240
VPS = VPAD // NS

NODE_BLK = 1000
EDGE_BLK = 4000


def _node_mm_body(x_ref, win_ref, wn_ref, ws_ref, hv_ref, hs_ref):
    h = jnp.maximum(
        jnp.dot(x_ref[...], win_ref[...], preferred_element_type=jnp.float32), 0.0
    )
    hv_ref[...] = jnp.dot(h, wn_ref[...], preferred_element_type=jnp.float32)
    hs_ref[...] = jnp.dot(h, ws_ref[...], preferred_element_type=jnp.float32)


def _node_mm(x, w_in, w_n, w_s):
    return pl.pallas_call(
        _node_mm_body,
        grid=(V // NODE_BLK,),
        in_specs=[
            pl.BlockSpec((NODE_BLK, D), lambda i: (i, 0)),
            pl.BlockSpec((D, D), lambda i: (0, 0)),
            pl.BlockSpec((D, D), lambda i: (0, 0)),
            pl.BlockSpec((D, D), lambda i: (0, 0)),
        ],
        out_specs=[
            pl.BlockSpec((NODE_BLK, D), lambda i: (i, 0)),
            pl.BlockSpec((NODE_BLK, D), lambda i: (i, 0)),
        ],
        out_shape=[
            jax.ShapeDtypeStruct((V, D), jnp.float32),
            jax.ShapeDtypeStruct((V, D), jnp.float32),
        ],
    )(x, w_in, w_n, w_s)


def _edge_mm_body(ef2_ref, wlo_ref, whi_ref, he2_ref):
    lo = jnp.dot(ef2_ref[...], wlo_ref[...], preferred_element_type=jnp.float32)
    hi = jnp.dot(ef2_ref[...], whi_ref[...], preferred_element_type=jnp.float32)
    au = jax.lax.bitcast_convert_type(
        lo.astype(jnp.bfloat16), jnp.uint16).astype(jnp.int32)
    bu = jax.lax.bitcast_convert_type(
        hi.astype(jnp.bfloat16), jnp.uint16).astype(jnp.int32)
    he2_ref[...] = au | (bu << 16)


def _edge_mm(ef2, w_lo, w_hi, blk):
    rows = ef2.shape[0]
    return pl.pallas_call(
        _edge_mm_body,
        grid=(rows // blk,),
        in_specs=[
            pl.BlockSpec((blk, 2 * D_EDGE), lambda i: (i, 0)),
            pl.BlockSpec((2 * D_EDGE, D), lambda i: (0, 0)),
            pl.BlockSpec((2 * D_EDGE, D), lambda i: (0, 0)),
        ],
        out_specs=pl.BlockSpec((blk, D), lambda i: (i, 0)),
        out_shape=jax.ShapeDtypeStruct((rows, D), jnp.int32),
    )(ef2, w_lo, w_hi)


def _packed_edge_weights(w_e):
    cols = jnp.arange(D).reshape(D // 32, 2, 16)
    w_a = w_e[:, cols[:, 0, :].reshape(-1)]
    w_b = w_e[:, cols[:, 1, :].reshape(-1)]
    z = jnp.zeros((D_EDGE, D // 2), jnp.float32)
    w_lo = jnp.concatenate(
        [jnp.concatenate([w_a, z], 1), jnp.concatenate([z, w_a], 1)], 0)
    w_hi = jnp.concatenate(
        [jnp.concatenate([w_b, z], 1), jnp.concatenate([z, w_b], 1)], 0)
    return w_lo, w_hi


def _edge_sc_body(nch, hv_hbm, idx_hbm, he2_hbm, init_hbm, out_hbm,
                  idxv, grows, erows, acc, si0, si1, si2, si3, si4,
                  sg0, sg1, sh0, sh1):
    c = lax.axis_index("c")
    s = lax.axis_index("s")
    wid = c * NS + s
    ept = nch * CHUNK

    zbase = s * VPS
    pltpu.sync_copy(init_hbm.at[c, pl.ds(zbase, VPS)],
                    acc.at[pl.ds(zbase, VPS)])
    plsc.subcore_barrier()

    ebase = wid * ept
    ebaseh = wid * (ept // 2)
    sis = (si0, si1, si2, si3, si4)
    sgs = (sg0, sg1)
    shs = (sh0, sh1)

    def _issue_idx(g, i):
        pltpu.async_copy(idx_hbm.at[wid, g], idxv.at[i], sis[i])

    def _wait_idx(i):
        pltpu.make_async_copy(idx_hbm.at[0, 0], idxv.at[i], sis[i]).wait()

    def _issue_data(g, i, b):
        pltpu.async_copy(hv_hbm.at[idxv.at[i, 0]], grows.at[b], sgs[b])
        pltpu.async_copy(
            he2_hbm.at[pl.ds(ebaseh + g * (CHUNK // 2), CHUNK // 2)],
            erows.at[b], shs[b])

    def _drain_data(b):
        pltpu.make_async_copy(hv_hbm.at[idxv.at[0, 0]], grows.at[b],
                              sgs[b]).wait()
        pltpu.make_async_copy(
            he2_hbm.at[pl.ds(ebaseh, CHUNK // 2)], erows.at[b],
            shs[b]).wait()

    def _mul_scatter(i, b):
        @plsc.parallel_loop(0, CHUNK // 2)
        def _mul(rp):
            for h in range(2):
                r = 2 * rp + h
                for m in range(D // 32):
                    w = erows[b, rp, pl.ds(h * 64 + m * 16, 16)]
                    af = jax.lax.bitcast_convert_type(w << 16, jnp.float32)
                    bf = jax.lax.bitcast_convert_type(
                        w & jnp.int32(-65536), jnp.float32)
                    sl0 = pl.ds((2 * m) * 16, 16)
                    sl1 = pl.ds((2 * m + 1) * 16, 16)
                    grows[b, r, sl0] = grows[b, r, sl0] * af
                    grows[b, r, sl1] = grows[b, r, sl1] * bf

        pltpu.sync_copy(grows.at[b], acc.at[idxv.at[i, 1]], add=True)

    tail = nch % UNROLL

    def _phase(g, base):
        i, b = g % NISLOT, g % 2
        if g + 1 < nch:
            _wait_idx((g + 1) % NISLOT)
            _issue_data(base + (g + 1), (g + 1) % NISLOT, (g + 1) % 2)
        _drain_data(b)
        _mul_scatter(i, b)
        if g + 4 < nch:
            _issue_idx(base + (g + 4), (g + 4) % NISLOT)

    for g in range(4):
        _issue_idx(g, g)
    _wait_idx(0)
    _issue_data(0, 0, 0)

    @pl.loop(0, (nch - tail) // UNROLL - 1)
    def _block(blk):
        base = blk * UNROLL
        for k in range(UNROLL):
            _phase(k, base)

    for g in range(nch - tail - UNROLL, nch):
        _phase(g, 0)

    plsc.subcore_barrier()
    pltpu.sync_copy(acc.at[pl.ds(zbase, VPS)], out_hbm.at[c, pl.ds(zbase, VPS)])


def _edge_sc(nch, hv, idx, he2, init):
    mesh = plsc.VectorSubcoreMesh(
        core_axis_name="c", subcore_axis_name="s", num_cores=NC, num_subcores=NS
    )
    return pl.kernel(
        functools.partial(_edge_sc_body, nch),
        out_type=jax.ShapeDtypeStruct((NC, VPAD, D), jnp.float32),
        mesh=mesh,
        scratch_types=[
            pltpu.VMEM((NISLOT, 2, CHUNK), jnp.int32),
            pltpu.VMEM((2, CHUNK, D), jnp.float32),
            pltpu.VMEM((2, CHUNK // 2, D), jnp.int32),
            pltpu.VMEM_SHARED((VPAD, D), jnp.float32),
            pltpu.SemaphoreType.DMA,
            pltpu.SemaphoreType.DMA,
            pltpu.SemaphoreType.DMA,
            pltpu.SemaphoreType.DMA,
            pltpu.SemaphoreType.DMA,
            pltpu.SemaphoreType.DMA,
            pltpu.SemaphoreType.DMA,
            pltpu.SemaphoreType.DMA,
            pltpu.SemaphoreType.DMA,
        ],
    )(hv, idx, he2, init)


def _combine_body(p_ref, hs_ref, out_ref):
    out_ref[...] = (p_ref[0] + p_ref[1]) * hs_ref[...]


def _combine(partials, h_self):
    return pl.pallas_call(
        _combine_body,
        grid=(V // NODE_BLK,),
        in_specs=[
            pl.BlockSpec((NC, NODE_BLK, D), lambda i: (0, i, 0)),
            pl.BlockSpec((NODE_BLK, D), lambda i: (i, 0)),
        ],
        out_specs=pl.BlockSpec((NODE_BLK, D), lambda i: (i, 0)),
        out_shape=jax.ShapeDtypeStruct((V, D), jnp.float32),
    )(partials, h_self)


def kernel(node_feats, edge_index, edge_feats, W_in, W_cm, b_cm, W_e, W_n, W_s):
    idx = jnp.stack([edge_index[0].reshape(NW, NCHUNK, CHUNK),
                     edge_index[1].reshape(NW, NCHUNK, CHUNK)], axis=2)
    hv, h_self = _node_mm(node_feats, W_in, W_n, W_s)
    w_lo, w_hi = _packed_edge_weights(W_e)
    ef2 = edge_feats.reshape(E // 2, 2 * D_EDGE)
    he2 = _edge_mm(ef2, w_lo, w_hi, 2000)
    zeros = jnp.zeros((NC, VPAD, D), jnp.float32)
    partials = _edge_sc(NCHUNK, hv, idx, he2, zeros)
    return _combine(partials, h_self)


# speedup vs baseline: 1.1780x; 1.0749x over previous
import functools

import jax
import jax.numpy as jnp
from jax import lax
from jax.experimental import pallas as pl
from jax.experimental.pallas import tpu as pltpu
from jax.experimental.pallas import tpu_sc as plsc

V = 10000
E = 320000
D = 128
D_EDGE = 16

NC = 2
NS = 16
NW = NC * NS
CHUNK = 80
NCHUNK = E // (NW * CHUNK)
NISLOT = 5
UNROLL = 10
VPAD = 10240
VPS = VPAD // NS

NODE_BLK = 1000
EDGE_BLK = 4000


def _node_mm_body(x_ref, win_ref, wn_ref, ws_ref, hv_ref, hs_ref):
    h = jnp.maximum(
        jnp.dot(x_ref[...], win_ref[...], preferred_element_type=jnp.float32), 0.0
    )
    hv_ref[...] = jnp.dot(h, wn_ref[...], preferred_element_type=jnp.float32)
    hs_ref[...] = jnp.dot(h, ws_ref[...], preferred_element_type=jnp.float32)


def _node_mm(x, w_in, w_n, w_s):
    return pl.pallas_call(
        _node_mm_body,
        grid=(V // NODE_BLK,),
        in_specs=[
            pl.BlockSpec((NODE_BLK, D), lambda i: (i, 0)),
            pl.BlockSpec((D, D), lambda i: (0, 0)),
            pl.BlockSpec((D, D), lambda i: (0, 0)),
            pl.BlockSpec((D, D), lambda i: (0, 0)),
        ],
        out_specs=[
            pl.BlockSpec((NODE_BLK, D), lambda i: (i, 0)),
            pl.BlockSpec((NODE_BLK, D), lambda i: (i, 0)),
        ],
        out_shape=[
            jax.ShapeDtypeStruct((V, D), jnp.float32),
            jax.ShapeDtypeStruct((V, D), jnp.float32),
        ],
    )(x, w_in, w_n, w_s)


def _edge_mm_body(ef2_ref, wlo_ref, whi_ref, he2_ref):
    lo = jnp.dot(ef2_ref[...], wlo_ref[...], preferred_element_type=jnp.float32)
    hi = jnp.dot(ef2_ref[...], whi_ref[...], preferred_element_type=jnp.float32)
    au = jax.lax.bitcast_convert_type(
        lo.astype(jnp.bfloat16), jnp.uint16).astype(jnp.int32)
    bu = jax.lax.bitcast_convert_type(
        hi.astype(jnp.bfloat16), jnp.uint16).astype(jnp.int32)
    he2_ref[...] = au | (bu << 16)


def _edge_mm(ef2, w_lo, w_hi, blk):
    rows = ef2.shape[0]
    return pl.pallas_call(
        _edge_mm_body,
        grid=(rows // blk,),
        in_specs=[
            pl.BlockSpec((blk, 2 * D_EDGE), lambda i: (i, 0)),
            pl.BlockSpec((2 * D_EDGE, D), lambda i: (0, 0)),
            pl.BlockSpec((2 * D_EDGE, D), lambda i: (0, 0)),
        ],
        out_specs=pl.BlockSpec((blk, D), lambda i: (i, 0)),
        out_shape=jax.ShapeDtypeStruct((rows, D), jnp.int32),
    )(ef2, w_lo, w_hi)


def _packed_edge_weights(w_e):
    cols = jnp.arange(D).reshape(D // 32, 2, 16)
    w_a = w_e[:, cols[:, 0, :].reshape(-1)]
    w_b = w_e[:, cols[:, 1, :].reshape(-1)]
    z = jnp.zeros((D_EDGE, D // 2), jnp.float32)
    w_lo = jnp.concatenate(
        [jnp.concatenate([w_a, z], 1), jnp.concatenate([z, w_a], 1)], 0)
    w_hi = jnp.concatenate(
        [jnp.concatenate([w_b, z], 1), jnp.concatenate([z, w_b], 1)], 0)
    return w_lo, w_hi


def _edge_sc_body(nch, hv_hbm, idx_hbm, he2_hbm, out_hbm,
                  idxv, grows, erows, acc, si0, si1, si2, si3, si4,
                  sg0, sg1, sh0, sh1):
    c = lax.axis_index("c")
    s = lax.axis_index("s")
    wid = c * NS + s
    ept = nch * CHUNK

    @plsc.parallel_loop(0, CHUNK)
    def _zero(r):
        for j in range(D // 16):
            grows[0, r, pl.ds(j * 16, 16)] = jnp.zeros((16,), jnp.float32)

    zbase = s * VPS
    for q in range(VPS // CHUNK):
        pltpu.sync_copy(grows.at[0],
                        acc.at[pl.ds(zbase + q * CHUNK, CHUNK)])
    plsc.subcore_barrier()

    ebase = wid * ept
    ebaseh = wid * (ept // 2)
    sis = (si0, si1, si2, si3, si4)
    sgs = (sg0, sg1)
    shs = (sh0, sh1)

    def _issue_idx(g, i):
        pltpu.async_copy(idx_hbm.at[wid, g], idxv.at[i], sis[i])

    def _wait_idx(i):
        pltpu.make_async_copy(idx_hbm.at[0, 0], idxv.at[i], sis[i]).wait()

    def _issue_data(g, i, b):
        pltpu.async_copy(hv_hbm.at[idxv.at[i, 0]], grows.at[b], sgs[b])
        pltpu.async_copy(
            he2_hbm.at[pl.ds(ebaseh + g * (CHUNK // 2), CHUNK // 2)],
            erows.at[b], shs[b])

    def _drain_data(b):
        pltpu.make_async_copy(hv_hbm.at[idxv.at[0, 0]], grows.at[b],
                              sgs[b]).wait()
        pltpu.make_async_copy(
            he2_hbm.at[pl.ds(ebaseh, CHUNK // 2)], erows.at[b],
            shs[b]).wait()

    def _mul_scatter(i, b):
        @plsc.parallel_loop(0, CHUNK // 2)
        def _mul(rp):
            for h in range(2):
                r = 2 * rp + h
                for m in range(D // 32):
                    w = erows[b, rp, pl.ds(h * 64 + m * 16, 16)]
                    af = jax.lax.bitcast_convert_type(w << 16, jnp.float32)
                    bf = jax.lax.bitcast_convert_type(
                        w & jnp.int32(-65536), jnp.float32)
                    sl0 = pl.ds((2 * m) * 16, 16)
                    sl1 = pl.ds((2 * m + 1) * 16, 16)
                    grows[b, r, sl0] = grows[b, r, sl0] * af
                    grows[b, r, sl1] = grows[b, r, sl1] * bf

        pltpu.sync_copy(grows.at[b], acc.at[idxv.at[i, 1]], add=True)

    tail = nch % UNROLL

    def _phase(g, base):
        i, b = g % NISLOT, g % 2
        if g + 1 < nch:
            _wait_idx((g + 1) % NISLOT)
            _issue_data(base + (g + 1), (g + 1) % NISLOT, (g + 1) % 2)
        _drain_data(b)
        _mul_scatter(i, b)
        if g + 4 < nch:
            _issue_idx(base + (g + 4), (g + 4) % NISLOT)

    for g in range(4):
        _issue_idx(g, g)
    _wait_idx(0)
    _issue_data(0, 0, 0)

    @pl.loop(0, (nch - tail) // UNROLL - 1)
    def _block(blk):
        base = blk * UNROLL
        for k in range(UNROLL):
            _phase(k, base)

    for g in range(nch - tail - UNROLL, nch):
        _phase(g, 0)

    plsc.subcore_barrier()
    pltpu.sync_copy(acc.at[pl.ds(zbase, VPS)], out_hbm.at[c, pl.ds(zbase, VPS)])


def _edge_sc(nch, hv, idx, he2):
    mesh = plsc.VectorSubcoreMesh(
        core_axis_name="c", subcore_axis_name="s", num_cores=NC, num_subcores=NS
    )
    return pl.kernel(
        functools.partial(_edge_sc_body, nch),
        out_type=jax.ShapeDtypeStruct((NC, VPAD, D), jnp.float32),
        mesh=mesh,
        scratch_types=[
            pltpu.VMEM((NISLOT, 2, CHUNK), jnp.int32),
            pltpu.VMEM((2, CHUNK, D), jnp.float32),
            pltpu.VMEM((2, CHUNK // 2, D), jnp.int32),
            pltpu.VMEM_SHARED((VPAD, D), jnp.float32),
            pltpu.SemaphoreType.DMA,
            pltpu.SemaphoreType.DMA,
            pltpu.SemaphoreType.DMA,
            pltpu.SemaphoreType.DMA,
            pltpu.SemaphoreType.DMA,
            pltpu.SemaphoreType.DMA,
            pltpu.SemaphoreType.DMA,
            pltpu.SemaphoreType.DMA,
            pltpu.SemaphoreType.DMA,
        ],
    )(hv, idx, he2)


def _combine_body(p_ref, hs_ref, out_ref):
    out_ref[...] = (p_ref[0] + p_ref[1]) * hs_ref[...]


def _combine(partials, h_self):
    return pl.pallas_call(
        _combine_body,
        grid=(V // NODE_BLK,),
        in_specs=[
            pl.BlockSpec((NC, NODE_BLK, D), lambda i: (0, i, 0)),
            pl.BlockSpec((NODE_BLK, D), lambda i: (i, 0)),
        ],
        out_specs=pl.BlockSpec((NODE_BLK, D), lambda i: (i, 0)),
        out_shape=jax.ShapeDtypeStruct((V, D), jnp.float32),
    )(partials, h_self)


def kernel(node_feats, edge_index, edge_feats, W_in, W_cm, b_cm, W_e, W_n, W_s):
    idx = jnp.stack([edge_index[0].reshape(NW, NCHUNK, CHUNK),
                     edge_index[1].reshape(NW, NCHUNK, CHUNK)], axis=2)
    hv, h_self = _node_mm(node_feats, W_in, W_n, W_s)
    w_lo, w_hi = _packed_edge_weights(W_e)
    ef2 = edge_feats.reshape(E // 2, 2 * D_EDGE)
    he2 = _edge_mm(ef2, w_lo, w_hi, 4000)
    partials = _edge_sc(NCHUNK, hv, idx, he2)
    return _combine(partials, h_self)


# speedup vs baseline: 1.1782x; 1.0001x over previous
import functools

import jax
import jax.numpy as jnp
from jax import lax
from jax.experimental import pallas as pl
from jax.experimental.pallas import tpu as pltpu
from jax.experimental.pallas import tpu_sc as plsc

V = 10000
E = 320000
D = 128
D_EDGE = 16

NC = 2
NS = 16
NW = NC * NS
CHUNK = 80
NCHUNK = E // (NW * CHUNK)
NISLOT = 5
UNROLL = 10
VPAD = 10240
VPS = VPAD // NS

NODE_BLK = 1000
EDGE_BLK = 4000


def _node_mm_body(x_ref, win_ref, wn_ref, ws_ref, hv_ref, hs_ref):
    h = jnp.maximum(
        jnp.dot(x_ref[...], win_ref[...], preferred_element_type=jnp.float32), 0.0
    )
    hv_ref[...] = jnp.dot(h, wn_ref[...], preferred_element_type=jnp.float32)
    hs_ref[...] = jnp.dot(h, ws_ref[...], preferred_element_type=jnp.float32)


def _node_mm(x, w_in, w_n, w_s):
    return pl.pallas_call(
        _node_mm_body,
        grid=(V // NODE_BLK,),
        in_specs=[
            pl.BlockSpec((NODE_BLK, D), lambda i: (i, 0)),
            pl.BlockSpec((D, D), lambda i: (0, 0)),
            pl.BlockSpec((D, D), lambda i: (0, 0)),
            pl.BlockSpec((D, D), lambda i: (0, 0)),
        ],
        out_specs=[
            pl.BlockSpec((NODE_BLK, D), lambda i: (i, 0)),
            pl.BlockSpec((NODE_BLK, D), lambda i: (i, 0)),
        ],
        out_shape=[
            jax.ShapeDtypeStruct((V, D), jnp.float32),
            jax.ShapeDtypeStruct((V, D), jnp.float32),
        ],
    )(x, w_in, w_n, w_s)


def _edge_mm_body(ef2_ref, wlo_ref, whi_ref, he2_ref):
    lo = jnp.dot(ef2_ref[...], wlo_ref[...], preferred_element_type=jnp.float32)
    hi = jnp.dot(ef2_ref[...], whi_ref[...], preferred_element_type=jnp.float32)
    au = jax.lax.bitcast_convert_type(
        lo.astype(jnp.bfloat16), jnp.uint16).astype(jnp.int32)
    bu = jax.lax.bitcast_convert_type(
        hi.astype(jnp.bfloat16), jnp.uint16).astype(jnp.int32)
    he2_ref[...] = au | (bu << 16)


def _edge_mm(ef2, w_lo, w_hi, blk):
    rows = ef2.shape[0]
    return pl.pallas_call(
        _edge_mm_body,
        grid=(rows // blk,),
        in_specs=[
            pl.BlockSpec((blk, 2 * D_EDGE), lambda i: (i, 0)),
            pl.BlockSpec((2 * D_EDGE, D), lambda i: (0, 0)),
            pl.BlockSpec((2 * D_EDGE, D), lambda i: (0, 0)),
        ],
        out_specs=pl.BlockSpec((blk, D), lambda i: (i, 0)),
        out_shape=jax.ShapeDtypeStruct((rows, D), jnp.int32),
    )(ef2, w_lo, w_hi)


def _packed_edge_weights(w_e):
    cols = jnp.arange(D).reshape(D // 32, 2, 16)
    w_a = w_e[:, cols[:, 0, :].reshape(-1)]
    w_b = w_e[:, cols[:, 1, :].reshape(-1)]
    z = jnp.zeros((D_EDGE, D // 2), jnp.float32)
    w_lo = jnp.concatenate(
        [jnp.concatenate([w_a, z], 1), jnp.concatenate([z, w_a], 1)], 0)
    w_hi = jnp.concatenate(
        [jnp.concatenate([w_b, z], 1), jnp.concatenate([z, w_b], 1)], 0)
    return w_lo, w_hi


def _edge_sc_body(nch, hv_hbm, idx_hbm, he2_hbm, out_hbm,
                  idxv, grows, erows, acc, si0, si1, si2, si3, si4,
                  sg0, sg1, sh0, sh1):
    c = lax.axis_index("c")
    s = lax.axis_index("s")
    wid = c * NS + s
    ept = nch * CHUNK

    @plsc.parallel_loop(0, CHUNK)
    def _zero(r):
        for j in range(D // 16):
            grows[0, r, pl.ds(j * 16, 16)] = jnp.zeros((16,), jnp.float32)

    zbase = s * VPS
    for q in range(VPS // CHUNK):
        pltpu.sync_copy(grows.at[0],
                        acc.at[pl.ds(zbase + q * CHUNK, CHUNK)])
    plsc.subcore_barrier()

    ebase = wid * ept
    ebaseh = wid * (ept // 2)
    sis = (si0, si1, si2, si3, si4)
    sgs = (sg0, sg1)
    shs = (sh0, sh1)

    def _issue_idx(g, i):
        pltpu.async_copy(idx_hbm.at[wid, g], idxv.at[i], sis[i])

    def _wait_idx(i):
        pltpu.make_async_copy(idx_hbm.at[0, 0], idxv.at[i], sis[i]).wait()

    def _issue_data(g, i, b):
        pltpu.async_copy(hv_hbm.at[idxv.at[i, 0]], grows.at[b], sgs[b])
        pltpu.async_copy(
            he2_hbm.at[pl.ds(ebaseh + g * (CHUNK // 2), CHUNK // 2)],
            erows.at[b], shs[b])

    def _drain_data(b):
        pltpu.make_async_copy(hv_hbm.at[idxv.at[0, 0]], grows.at[b],
                              sgs[b]).wait()
        pltpu.make_async_copy(
            he2_hbm.at[pl.ds(ebaseh, CHUNK // 2)], erows.at[b],
            shs[b]).wait()

    def _mul_scatter(i, b):
        @plsc.parallel_loop(0, CHUNK // 2)
        def _mul(rp):
            for h in range(2):
                r = 2 * rp + h
                for m in range(D // 32):
                    w = erows[b, rp, pl.ds(h * 64 + m * 16, 16)]
                    af = jax.lax.bitcast_convert_type(w << 16, jnp.float32)
                    bf = jax.lax.bitcast_convert_type(
                        w & jnp.int32(-65536), jnp.float32)
                    sl0 = pl.ds((2 * m) * 16, 16)
                    sl1 = pl.ds((2 * m + 1) * 16, 16)
                    grows[b, r, sl0] = grows[b, r, sl0] * af
                    grows[b, r, sl1] = grows[b, r, sl1] * bf

        pltpu.sync_copy(grows.at[b], acc.at[idxv.at[i, 1]], add=True)

    tail = nch % UNROLL

    def _phase(g, base):
        i, b = g % NISLOT, g % 2
        if g + 1 < nch:
            _wait_idx((g + 1) % NISLOT)
            _issue_data(base + (g + 1), (g + 1) % NISLOT, (g + 1) % 2)
        _drain_data(b)
        _mul_scatter(i, b)
        if g + 4 < nch:
            _issue_idx(base + (g + 4), (g + 4) % NISLOT)

    for g in range(4):
        _issue_idx(g, g)
    _wait_idx(0)
    _issue_data(0, 0, 0)

    @pl.loop(0, (nch - tail) // UNROLL - 1)
    def _block(blk):
        base = blk * UNROLL
        for k in range(UNROLL):
            _phase(k, base)

    for g in range(nch - tail - UNROLL, nch):
        _phase(g, 0)

    plsc.subcore_barrier()
    pltpu.sync_copy(acc.at[pl.ds(zbase, VPS)], out_hbm.at[c, pl.ds(zbase, VPS)])


def _edge_sc(nch, hv, idx, he2):
    mesh = plsc.VectorSubcoreMesh(
        core_axis_name="c", subcore_axis_name="s", num_cores=NC, num_subcores=NS
    )
    return pl.kernel(
        functools.partial(_edge_sc_body, nch),
        out_type=jax.ShapeDtypeStruct((NC, VPAD, D), jnp.float32),
        mesh=mesh,
        scratch_types=[
            pltpu.VMEM((NISLOT, 2, CHUNK), jnp.int32),
            pltpu.VMEM((2, CHUNK, D), jnp.float32),
            pltpu.VMEM((2, CHUNK // 2, D), jnp.int32),
            pltpu.VMEM_SHARED((VPAD, D), jnp.float32),
            pltpu.SemaphoreType.DMA,
            pltpu.SemaphoreType.DMA,
            pltpu.SemaphoreType.DMA,
            pltpu.SemaphoreType.DMA,
            pltpu.SemaphoreType.DMA,
            pltpu.SemaphoreType.DMA,
            pltpu.SemaphoreType.DMA,
            pltpu.SemaphoreType.DMA,
            pltpu.SemaphoreType.DMA,
        ],
    )(hv, idx, he2)


def _combine_body(p_ref, hs_ref, out_ref):
    out_ref[...] = (p_ref[0] + p_ref[1]) * hs_ref[...]


def _combine(partials, h_self):
    return pl.pallas_call(
        _combine_body,
        grid=(V // NODE_BLK,),
        in_specs=[
            pl.BlockSpec((NC, NODE_BLK, D), lambda i: (0, i, 0)),
            pl.BlockSpec((NODE_BLK, D), lambda i: (i, 0)),
        ],
        out_specs=pl.BlockSpec((NODE_BLK, D), lambda i: (i, 0)),
        out_shape=jax.ShapeDtypeStruct((V, D), jnp.float32),
    )(partials, h_self)


def kernel(node_feats, edge_index, edge_feats, W_in, W_cm, b_cm, W_e, W_n, W_s):
    idx = jnp.stack([edge_index[0].reshape(NW, NCHUNK, CHUNK),
                     edge_index[1].reshape(NW, NCHUNK, CHUNK)], axis=2)
    hv, h_self = _node_mm(node_feats, W_in, W_n, W_s)
    w_lo, w_hi = _packed_edge_weights(W_e)
    ef2 = edge_feats.reshape(E // 2, 2 * D_EDGE)
    he2 = _edge_mm(ef2, w_lo, w_hi, EDGE_BLK)
    partials = _edge_sc(NCHUNK, hv, idx, he2)
    return _combine(partials, h_self)
